# baseline (speedup 1.0000x reference)
import functools

import jax
import jax.numpy as jnp
from jax import lax
from jax.experimental import pallas as pl
from jax.experimental.pallas import tpu as pltpu
from jax.experimental.pallas import tpu_sc as plsc

NUM_CAT = 26
N_FIELDS = 39
N_NUM = N_FIELDS - NUM_CAT
DIM = 128
BATCH = 16384

NC, NS = 2, 16
NW = NC * NS
BT = BATCH * NUM_CAT
STAGE_ROWS = BATCH * N_FIELDS
PW = BT // NW
CH = 128
NCH = PW // CH
GRP = 2
NGRP = NCH // GRP
NBUF = 2
NITER = NGRP // NBUF

_sc_mesh = plsc.VectorSubcoreMesh(core_axis_name="c", subcore_axis_name="s")


@functools.partial(
    pl.kernel,
    out_type=jax.ShapeDtypeStruct((STAGE_ROWS, DIM), jnp.float32),
    mesh=_sc_mesh,
    scratch_types=[
        pltpu.VMEM_SHARED((256, DIM), jnp.float32),
        pltpu.VMEM((NCH, CH), jnp.int32),
    ]
    + [pltpu.VMEM((GRP * CH, DIM), jnp.float32) for _ in range(NBUF)]
    + [pltpu.SemaphoreType.DMA for _ in range(2 * NBUF)],
)
def _sc_gather(table_hbm, idx_hbm, out_hbm, tab_sh, idx_v, *bufs_sems):
    rows = bufs_sems[:NBUF]
    gsem = bufs_sems[NBUF : 2 * NBUF]
    wsem = bufs_sems[2 * NBUF :]
    wid = lax.axis_index("s") * NC + lax.axis_index("c")
    base = wid * PW
    GROWS = GRP * CH
    pltpu.sync_copy(idx_hbm.at[wid], idx_v)

    @pl.when(lax.axis_index("s") == 0)
    def _():
        pltpu.sync_copy(table_hbm, tab_sh)

    plsc.subcore_barrier()

    def body(i, carry):
        for j in range(NBUF):
            g = i * NBUF + j

            @pl.when(i > 0)
            def _():
                pltpu.make_async_copy(
                    rows[j],
                    out_hbm.at[pl.ds(base + (g - NBUF) * GROWS, GROWS)],
                    wsem[j],
                ).wait()

            for k in range(GRP):
                pltpu.async_copy(
                    tab_sh.at[idx_v.at[g * GRP + k]],
                    rows[j].at[pl.ds(k * CH, CH)],
                    gsem[j],
                )
        for j in range(NBUF):
            g = i * NBUF + j
            for k in range(GRP):
                pltpu.make_async_copy(
                    tab_sh.at[idx_v.at[g * GRP + k]],
                    rows[j].at[pl.ds(k * CH, CH)],
                    gsem[j],
                ).wait()
            pltpu.async_copy(
                rows[j], out_hbm.at[pl.ds(base + g * GROWS, GROWS)], wsem[j]
            )
        return carry

    lax.fori_loop(0, NITER, body, 0)
    for j in range(NBUF):
        g = NGRP - NBUF + j
        pltpu.make_async_copy(
            rows[j], out_hbm.at[pl.ds(base + g * GROWS, GROWS)], wsem[j]
        ).wait()


_NB = BATCH // 2


def _numfill_body(src_ref, num_ref, w_ref, b_ref, out_ref):
    del src_ref
    i = pl.program_id(0)
    field = i // 2
    half = i % 2
    numv = num_ref[field, pl.ds(half * _NB, _NB)]
    w = w_ref[0, :]
    b = b_ref[0, :]
    out_ref[...] = jnp.maximum(numv[:, None] * w[None, :] + b[None, :], 0.0)


def _numfill(stage, numT, w, b):
    grid = 2 * N_NUM
    return pl.pallas_call(
        _numfill_body,
        grid=(grid,),
        in_specs=[
            pl.BlockSpec(memory_space=pl.ANY),
            pl.BlockSpec((N_NUM, BATCH), lambda i: (0, 0)),
            pl.BlockSpec((1, DIM), lambda i: (0, 0)),
            pl.BlockSpec((1, DIM), lambda i: (0, 0)),
        ],
        out_specs=pl.BlockSpec((_NB, DIM), lambda i: (i + 2 * NUM_CAT, 0)),
        out_shape=jax.ShapeDtypeStruct((STAGE_ROWS, DIM), jnp.float32),
        input_output_aliases={0: 0},
    )(stage, numT, w, b)


def kernel(x, table, W_num, b_num):
    xT = x.T
    idxT = xT[:NUM_CAT].astype(jnp.int32).reshape(NW, NCH, CH)
    numT = xT[NUM_CAT:]
    stage = _sc_gather(table, idxT)
    full = _numfill(stage, numT, W_num.reshape(1, DIM), b_num.reshape(1, DIM))
    out3 = full.reshape(N_FIELDS, BATCH, DIM)
    return jnp.transpose(out3, (1, 0, 2))


# speedup vs baseline: 1.3178x; 1.3178x over previous
import functools

import jax
import jax.numpy as jnp
from jax import lax
from jax.experimental import pallas as pl
from jax.experimental.pallas import tpu as pltpu
from jax.experimental.pallas import tpu_sc as plsc

NUM_CAT = 26
N_FIELDS = 39
N_NUM = N_FIELDS - NUM_CAT
DIM = 128
BATCH = 16384

NC, NS = 2, 16
NW = NC * NS
BT = BATCH * NUM_CAT
STAGE_ROWS = BATCH * N_FIELDS
PW = BT // NW
CH = 128
NCH = PW // CH
NBUF = 4
NITER = NCH // NBUF

_sc_mesh = plsc.VectorSubcoreMesh(core_axis_name="c", subcore_axis_name="s")


@functools.partial(
    pl.kernel,
    out_type=jax.ShapeDtypeStruct((STAGE_ROWS, DIM), jnp.float32),
    mesh=_sc_mesh,
    scratch_types=[
        pltpu.VMEM_SHARED((256, DIM), jnp.float32),
        pltpu.VMEM((NCH, CH), jnp.int32),
    ]
    + [pltpu.VMEM((CH, DIM), jnp.float32) for _ in range(NBUF)]
    + [pltpu.SemaphoreType.DMA for _ in range(2 * NBUF)],
)
def _sc_gather(table_hbm, idx_hbm, out_hbm, tab_sh, idx_v, *bufs_sems):
    rows = bufs_sems[:NBUF]
    gsem = bufs_sems[NBUF : 2 * NBUF]
    wsem = bufs_sems[2 * NBUF :]
    wid = lax.axis_index("s") * NC + lax.axis_index("c")
    base = wid * PW
    pltpu.sync_copy(idx_hbm.at[wid], idx_v)

    @pl.when(lax.axis_index("s") == 0)
    def _():
        pltpu.sync_copy(table_hbm, tab_sh)

    plsc.subcore_barrier()

    def body(i, carry):
        for j in range(NBUF):
            c = i * NBUF + j

            @pl.when(i > 0)
            def _():
                pltpu.make_async_copy(
                    rows[j], out_hbm.at[pl.ds(base + (c - NBUF) * CH, CH)], wsem[j]
                ).wait()

            pltpu.async_copy(tab_sh.at[idx_v.at[c]], rows[j], gsem[j])
        for j in range(NBUF):
            c = i * NBUF + j
            pltpu.make_async_copy(tab_sh.at[idx_v.at[c]], rows[j], gsem[j]).wait()
            pltpu.async_copy(rows[j], out_hbm.at[pl.ds(base + c * CH, CH)], wsem[j])
        return carry

    lax.fori_loop(0, NITER, body, 0)
    for j in range(NBUF):
        c = NCH - NBUF + j
        pltpu.make_async_copy(
            rows[j], out_hbm.at[pl.ds(base + c * CH, CH)], wsem[j]
        ).wait()


def _numfill_body(src_ref, num_ref, w_ref, b_ref, out_ref):
    del src_ref
    i = pl.program_id(0)
    numv = num_ref[i, :]
    w = w_ref[0, :]
    b = b_ref[0, :]
    out_ref[...] = jnp.maximum(numv[:, None] * w[None, :] + b[None, :], 0.0)


def _numfill(stage, numT, w, b):
    return pl.pallas_call(
        _numfill_body,
        grid=(N_NUM,),
        in_specs=[
            pl.BlockSpec(memory_space=pl.ANY),
            pl.BlockSpec((N_NUM, BATCH), lambda i: (0, 0)),
            pl.BlockSpec((1, DIM), lambda i: (0, 0)),
            pl.BlockSpec((1, DIM), lambda i: (0, 0)),
        ],
        out_specs=pl.BlockSpec((BATCH, DIM), lambda i: (i + NUM_CAT, 0)),
        out_shape=jax.ShapeDtypeStruct((STAGE_ROWS, DIM), jnp.float32),
        input_output_aliases={0: 0},
    )(stage, numT, w, b)


def kernel(x, table, W_num, b_num):
    xT = x.T
    idxT = xT[:NUM_CAT].astype(jnp.int32).reshape(NW, NCH, CH)
    numT = xT[NUM_CAT:]
    stage = _sc_gather(table, idxT)
    full = _numfill(stage, numT, W_num.reshape(1, DIM), b_num.reshape(1, DIM))
    out3 = full.reshape(N_FIELDS, BATCH, DIM)
    return jnp.transpose(out3, (1, 0, 2))
